# baseline (speedup 1.0000x reference)
import functools

import numpy as np
import jax
import jax.numpy as jnp
from jax import lax
from jax.experimental import pallas as pl
from jax.experimental.pallas import tpu as pltpu
from jax.experimental.pallas import tpu_sc as plsc

V = 10000
VP = 10240
D = 128
H = 64
E = 320000
NC = 2
NS = 16
NW = NC * NS
EPW = E // NW
CH = 80
NCHUNK = EPW // CH
NB = 10
RPS = VP // NS

BR = 1280
GRID = VP // BR

_RHO = np.empty(H, dtype=np.int32)
for _g in range(H // 32):
    for _i in range(16):
        _RHO[32 * _g + _i] = 32 * _g + 2 * _i
        _RHO[32 * _g + 16 + _i] = 32 * _g + 2 * _i + 1
_P_UNPERM = np.zeros((H, H), dtype=np.float32)
_P_UNPERM[np.arange(H), _RHO] = 1.0


def _mm(x, w):
    return jnp.dot(x, w, preferred_element_type=jnp.float32)


def _t0_body(x_ref, wi_ref, wm_ref, h_ref, a_ref):
    h = jnp.tanh(_mm(x_ref[...], wi_ref[...]))
    h_ref[...] = h
    a_ref[...] = jnp.maximum(_mm(h, wm_ref[...]), 0.0).astype(jnp.bfloat16)


def _t_dense_body(p_ref, wd_ref, wm_ref, a_ref):
    s = p_ref[0] + p_ref[1]
    c = jnp.tanh(_mm(s, wd_ref[...]))
    a_ref[...] = jnp.maximum(_mm(c, wm_ref[...]), 0.0).astype(jnp.bfloat16)


def _t_res_body(p_ref, h_ref, wm_ref, a_ref):
    m = (p_ref[0] + p_ref[1] + h_ref[...]) * 0.5
    a_ref[...] = jnp.maximum(_mm(m, wm_ref[...]), 0.0).astype(jnp.bfloat16)


def _t_sum_body(p_ref, unperm_ref, o_ref):
    o_ref[...] = _mm(p_ref[0] + p_ref[1], unperm_ref[...])


_F = jax.ShapeDtypeStruct
_W_SPEC = pl.BlockSpec((H, H), lambda i: (0, 0))
_ROW_SPEC = pl.BlockSpec((BR, H), lambda i: (i, 0))
_P_SPEC = pl.BlockSpec((NC, BR, H), lambda i: (0, i, 0))


def _tc0(x, wi, wm):
    return pl.pallas_call(
        _t0_body,
        grid=(GRID,),
        in_specs=[pl.BlockSpec((BR, D), lambda i: (i, 0)),
                  pl.BlockSpec((D, H), lambda i: (0, 0)), _W_SPEC],
        out_specs=(_ROW_SPEC, _ROW_SPEC),
        out_shape=(_F((VP, H), jnp.float32), _F((VP, H), jnp.bfloat16)),
    )(x, wi, wm)


def _tc_dense(p, wd, wm):
    return pl.pallas_call(
        _t_dense_body,
        grid=(GRID,),
        in_specs=[_P_SPEC, _W_SPEC, _W_SPEC],
        out_specs=_ROW_SPEC,
        out_shape=_F((VP, H), jnp.bfloat16),
    )(p, wd, wm)


def _tc_res(p, h, wm):
    return pl.pallas_call(
        _t_res_body,
        grid=(GRID,),
        in_specs=[_P_SPEC, _ROW_SPEC, _W_SPEC],
        out_specs=_ROW_SPEC,
        out_shape=_F((VP, H), jnp.bfloat16),
    )(p, h, wm)


def _tc_sum(p, unperm):
    return pl.pallas_call(
        _t_sum_body,
        grid=(V // 2000,),
        in_specs=[pl.BlockSpec((NC, 2000, H), lambda i: (0, i, 0)), _W_SPEC],
        out_specs=pl.BlockSpec((2000, H), lambda i: (i, 0)),
        out_shape=_F((V, H), jnp.float32),
    )(p, unperm)


_mesh = plsc.VectorSubcoreMesh(core_axis_name="c", subcore_axis_name="s")


@functools.partial(
    pl.kernel,
    out_type=_F((NC, VP, H), jnp.float32),
    mesh=_mesh,
    scratch_types=[
        pltpu.VMEM((NCHUNK, CH), jnp.int32),
        pltpu.VMEM((NCHUNK, CH), jnp.int32),
        pltpu.VMEM((NB, CH, H), jnp.bfloat16),
        pltpu.VMEM((CH, H), jnp.float32),
        pltpu.VMEM_SHARED((VP, H), jnp.float32),
        pltpu.SemaphoreType.DMA((NB,)),
    ],
    compiler_params=pltpu.CompilerParams(use_tc_tiling_on_sc=False,
                                         needs_layout_passes=False),
)
def _sc_edge_pass(a_hbm, src_hbm, dst_hbm, z_hbm, out_hbm,
                  srcv, dstv, rows, wide, acc, gsem):
    c = lax.axis_index("c")
    s = lax.axis_index("s")
    wid = c * NS + s

    pltpu.sync_copy(z_hbm.at[pl.ds(s * RPS, RPS)],
                    acc.at[pl.ds(s * RPS, RPS)])
    pltpu.sync_copy(src_hbm.at[wid], srcv)
    pltpu.sync_copy(dst_hbm.at[wid], dstv)
    plsc.subcore_barrier()

    def _start_gather(j, b):
        pltpu.async_copy(a_hbm.at[srcv.at[j]], rows.at[b], gsem.at[b])

    def _wait_gather(b):
        pltpu.make_async_copy(
            a_hbm.at[srcv.at[0]], rows.at[b], gsem.at[b]).wait()

    def _scatter(j, b):
        rb = rows.at[b]

        @pl.loop(0, CH, step=8)
        def _(r0):
            for rr in range(8):
                for g in range(H // 32):
                    u, v = plsc.unpack(rb[r0 + rr, pl.ds(32 * g, 32)],
                                       format=plsc.PackFormat.INTERLEAVED)
                    wide[r0 + rr, pl.ds(32 * g, 16)] = u
                    wide[r0 + rr, pl.ds(32 * g + 16, 16)] = v

        pltpu.sync_copy(wide, acc.at[dstv.at[j]], add=True)

    for b in range(NB):
        _start_gather(b, b)

    _MAIN = (NCHUNK // NB) * NB
    _TAIL = NCHUNK - _MAIN

    @pl.loop(0, _MAIN, step=NB)
    def _(j):
        for b in range(NB):
            _wait_gather(b)
            _scatter(j + b, b)

            @pl.when(j + b + NB < NCHUNK)
            def _():
                _start_gather(j + b + NB, b)

    for b in range(_TAIL):
        _wait_gather(b)
        _scatter(_MAIN + b, b)

    plsc.subcore_barrier()
    pltpu.sync_copy(acc.at[pl.ds(s * RPS, RPS)],
                    out_hbm.at[c, pl.ds(s * RPS, RPS)])


def kernel(node_features, adjacency_list_0, node_to_graph_map, num_graphs,
           W_init, W_mp0, W_mp1, W_mp2, W_mp3, W_dense0, W_dense2):
    src3 = adjacency_list_0[:, 0].reshape(NW, NCHUNK, CH)
    dst3 = adjacency_list_0[:, 1].reshape(NW, NCHUNK, CH)
    zeros = jnp.zeros((VP, H), jnp.float32)
    xpad = jnp.pad(node_features, ((0, VP - V), (0, 0)))

    rho = jnp.asarray(_RHO)
    unperm = jnp.asarray(_P_UNPERM)
    wi_p = W_init[:, rho]
    wmp0_p = W_mp0[rho, :]
    wd0_p = W_dense0[rho, :]
    wmp2_p = W_mp2[rho, :]
    wd2_p = W_dense2[rho, :]

    def edge_pass(a):
        return _sc_edge_pass(a, src3, dst3, zeros)

    h0, a0 = _tc0(xpad, wi_p, wmp0_p)
    p0 = edge_pass(a0)
    a1 = _tc_dense(p0, wd0_p, W_mp1)
    p1 = edge_pass(a1)
    a2 = _tc_res(p1, h0, wmp2_p)
    p2 = edge_pass(a2)
    a3 = _tc_dense(p2, wd2_p, W_mp3)
    p3 = edge_pass(a3)
    return _tc_sum(p3, unperm)


# speedup vs baseline: 1.0006x; 1.0006x over previous
import functools

import numpy as np
import jax
import jax.numpy as jnp
from jax import lax
from jax.experimental import pallas as pl
from jax.experimental.pallas import tpu as pltpu
from jax.experimental.pallas import tpu_sc as plsc

V = 10000
VP = 10240
D = 128
H = 64
E = 320000
NC = 2
NS = 16
NW = NC * NS
EPW = E // NW
CH = 80
NCHUNK = EPW // CH
NB = 10
RPS = VP // NS

BR = 1280
GRID = VP // BR

_RHO = np.empty(H, dtype=np.int32)
for _g in range(H // 32):
    for _i in range(16):
        _RHO[32 * _g + _i] = 32 * _g + 2 * _i
        _RHO[32 * _g + 16 + _i] = 32 * _g + 2 * _i + 1
_P_UNPERM = np.zeros((H, H), dtype=np.float32)
_P_UNPERM[np.arange(H), _RHO] = 1.0


def _mm(x, w):
    return jnp.dot(x, w, preferred_element_type=jnp.float32)


def _t0_body(x_ref, wi_ref, wm_ref, h_ref, a_ref):
    h = jnp.tanh(_mm(x_ref[...], wi_ref[...]))
    h_ref[...] = h
    a_ref[...] = jnp.maximum(_mm(h, wm_ref[...]), 0.0).astype(jnp.bfloat16)


def _t_dense_body(p_ref, wd_ref, wm_ref, a_ref):
    s = p_ref[0] + p_ref[1]
    c = jnp.tanh(_mm(s, wd_ref[...]))
    a_ref[...] = jnp.maximum(_mm(c, wm_ref[...]), 0.0).astype(jnp.bfloat16)


def _t_res_body(p_ref, h_ref, wm_ref, a_ref):
    m = (p_ref[0] + p_ref[1] + h_ref[...]) * 0.5
    a_ref[...] = jnp.maximum(_mm(m, wm_ref[...]), 0.0).astype(jnp.bfloat16)


def _t_sum_body(p_ref, unperm_ref, o_ref):
    o_ref[...] = _mm(p_ref[0] + p_ref[1], unperm_ref[...])


_F = jax.ShapeDtypeStruct
_W_SPEC = pl.BlockSpec((H, H), lambda i: (0, 0))
_ROW_SPEC = pl.BlockSpec((BR, H), lambda i: (i, 0))
_P_SPEC = pl.BlockSpec((NC, BR, H), lambda i: (0, i, 0))


def _tc0(x, wi, wm):
    return pl.pallas_call(
        _t0_body,
        grid=(GRID,),
        in_specs=[pl.BlockSpec((BR, D), lambda i: (i, 0)),
                  pl.BlockSpec((D, H), lambda i: (0, 0)), _W_SPEC],
        out_specs=(_ROW_SPEC, _ROW_SPEC),
        out_shape=(_F((VP, H), jnp.float32), _F((VP, H), jnp.bfloat16)),
    )(x, wi, wm)


def _tc_dense(p, wd, wm):
    return pl.pallas_call(
        _t_dense_body,
        grid=(GRID,),
        in_specs=[_P_SPEC, _W_SPEC, _W_SPEC],
        out_specs=_ROW_SPEC,
        out_shape=_F((VP, H), jnp.bfloat16),
    )(p, wd, wm)


def _tc_res(p, h, wm):
    return pl.pallas_call(
        _t_res_body,
        grid=(GRID,),
        in_specs=[_P_SPEC, _ROW_SPEC, _W_SPEC],
        out_specs=_ROW_SPEC,
        out_shape=_F((VP, H), jnp.bfloat16),
    )(p, h, wm)


def _tc_sum(p, unperm):
    return pl.pallas_call(
        _t_sum_body,
        grid=(V // 2000,),
        in_specs=[pl.BlockSpec((NC, 2000, H), lambda i: (0, i, 0)), _W_SPEC],
        out_specs=pl.BlockSpec((2000, H), lambda i: (i, 0)),
        out_shape=_F((V, H), jnp.float32),
    )(p, unperm)


_mesh = plsc.VectorSubcoreMesh(core_axis_name="c", subcore_axis_name="s")


@functools.partial(
    pl.kernel,
    out_type=_F((NC, VP, H), jnp.float32),
    mesh=_mesh,
    scratch_types=[
        pltpu.VMEM((NCHUNK, CH), jnp.int32),
        pltpu.VMEM((NCHUNK, CH), jnp.int32),
        pltpu.VMEM((NB, CH, H), jnp.bfloat16),
        pltpu.VMEM((CH, H), jnp.float32),
        pltpu.VMEM_SHARED((VP, H), jnp.float32),
        pltpu.SemaphoreType.DMA((NB,)),
    ],
    compiler_params=pltpu.CompilerParams(use_tc_tiling_on_sc=False,
                                         needs_layout_passes=False),
)
def _sc_edge_pass(a_hbm, src_hbm, dst_hbm, z_hbm, out_hbm,
                  srcv, dstv, rows, wide, acc, gsem):
    c = lax.axis_index("c")
    s = lax.axis_index("s")
    wid = c * NS + s

    pltpu.sync_copy(z_hbm.at[pl.ds(s * RPS, RPS)],
                    acc.at[pl.ds(s * RPS, RPS)])
    pltpu.sync_copy(src_hbm.at[wid], srcv)
    pltpu.sync_copy(dst_hbm.at[wid], dstv)
    plsc.subcore_barrier()

    def _start_gather(j, b):
        pltpu.async_copy(a_hbm.at[srcv.at[j]], rows.at[b], gsem.at[b])

    def _wait_gather(b):
        pltpu.make_async_copy(
            a_hbm.at[srcv.at[0]], rows.at[b], gsem.at[b]).wait()

    def _scatter(j, b):
        rb = rows.at[b]

        @pl.loop(0, CH, step=8)
        def _(r0):
            for rr in range(8):
                for g in range(H // 32):
                    pair = plsc.bitcast(rb[r0 + rr, pl.ds(32 * g, 32)],
                                        jnp.int32)
                    u = plsc.bitcast(pair << 16, jnp.float32)
                    v = plsc.bitcast(pair & jnp.int32(-65536), jnp.float32)
                    wide[r0 + rr, pl.ds(32 * g, 16)] = u
                    wide[r0 + rr, pl.ds(32 * g + 16, 16)] = v

        pltpu.sync_copy(wide, acc.at[dstv.at[j]], add=True)

    for b in range(NB):
        _start_gather(b, b)

    _MAIN = (NCHUNK // NB) * NB
    _TAIL = NCHUNK - _MAIN

    @pl.loop(0, _MAIN, step=NB)
    def _(j):
        for b in range(NB):
            _wait_gather(b)
            _scatter(j + b, b)

            @pl.when(j + b + NB < NCHUNK)
            def _():
                _start_gather(j + b + NB, b)

    for b in range(_TAIL):
        _wait_gather(b)
        _scatter(_MAIN + b, b)

    plsc.subcore_barrier()
    pltpu.sync_copy(acc.at[pl.ds(s * RPS, RPS)],
                    out_hbm.at[c, pl.ds(s * RPS, RPS)])


def kernel(node_features, adjacency_list_0, node_to_graph_map, num_graphs,
           W_init, W_mp0, W_mp1, W_mp2, W_mp3, W_dense0, W_dense2):
    src3 = adjacency_list_0[:, 0].reshape(NW, NCHUNK, CH)
    dst3 = adjacency_list_0[:, 1].reshape(NW, NCHUNK, CH)
    zeros = jnp.zeros((VP, H), jnp.float32)
    xpad = jnp.pad(node_features, ((0, VP - V), (0, 0)))

    rho = jnp.asarray(_RHO)
    unperm = jnp.asarray(_P_UNPERM)
    wi_p = W_init[:, rho]
    wmp0_p = W_mp0[rho, :]
    wd0_p = W_dense0[rho, :]
    wmp2_p = W_mp2[rho, :]
    wd2_p = W_dense2[rho, :]

    def edge_pass(a):
        return _sc_edge_pass(a, src3, dst3, zeros)

    h0, a0 = _tc0(xpad, wi_p, wmp0_p)
    p0 = edge_pass(a0)
    a1 = _tc_dense(p0, wd0_p, W_mp1)
    p1 = edge_pass(a1)
    a2 = _tc_res(p1, h0, wmp2_p)
    p2 = edge_pass(a2)
    a3 = _tc_dense(p2, wd2_p, W_mp3)
    p3 = edge_pass(a3)
    return _tc_sum(p3, unperm)


# speedup vs baseline: 2.5688x; 2.5672x over previous
import functools

import jax
import jax.numpy as jnp
from jax import lax
from jax.experimental import pallas as pl
from jax.experimental.pallas import tpu as pltpu
from jax.experimental.pallas import tpu_sc as plsc

V = 10000
VP = 10240
D = 128
H = 64
E = 320000
NC = 2
NS = 16
NW = NC * NS
EPW = E // NW
CH = 80
NCHUNK = EPW // CH
NB = 10
RPS = VP // NS

VPH = VP * H
PR = VP // 2


def _mm(x, w):
    return jnp.dot(x, w, preferred_element_type=jnp.float32)


def _pairs(ref):
    return ref[...].reshape(PR, 2 * H)


def _t0_body(x2_ref, wi_ref, wm_ref, h_ref, a_ref):
    h = jnp.tanh(_mm(x2_ref[...], wi_ref[...]))
    h_ref[...] = h.reshape(VPH)
    a_ref[...] = jnp.maximum(_mm(h, wm_ref[...]), 0.0).reshape(VPH)


def _t_dense_body(p0_ref, p1_ref, wd_ref, wm_ref, a_ref):
    s = _pairs(p0_ref) + _pairs(p1_ref)
    c = jnp.tanh(_mm(s, wd_ref[...]))
    a_ref[...] = jnp.maximum(_mm(c, wm_ref[...]), 0.0).reshape(VPH)


def _t_res_body(p0_ref, p1_ref, h_ref, wm_ref, a_ref):
    m = (_pairs(p0_ref) + _pairs(p1_ref) + _pairs(h_ref)) * 0.5
    a_ref[...] = jnp.maximum(_mm(m, wm_ref[...]), 0.0).reshape(VPH)


def _t_sum_body(p0_ref, p1_ref, o_ref):
    o_ref[...] = p0_ref[...] + p1_ref[...]


_F = jax.ShapeDtypeStruct


def _tc0(x2, wi2, wm2):
    return pl.pallas_call(
        _t0_body,
        out_shape=(_F((VPH,), jnp.float32), _F((VPH,), jnp.float32)),
    )(x2, wi2, wm2)


def _tc_dense(p0, p1, wd2, wm2):
    return pl.pallas_call(
        _t_dense_body, out_shape=_F((VPH,), jnp.float32))(p0, p1, wd2, wm2)


def _tc_res(p0, p1, h, wm2):
    return pl.pallas_call(
        _t_res_body, out_shape=_F((VPH,), jnp.float32))(p0, p1, h, wm2)


def _tc_sum(p0, p1):
    return pl.pallas_call(
        _t_sum_body, out_shape=_F((VPH,), jnp.float32))(p0, p1)


_mesh = plsc.VectorSubcoreMesh(core_axis_name="c", subcore_axis_name="s")


@functools.partial(
    pl.kernel,
    out_type=(_F((VP, H), jnp.float32), _F((VP, H), jnp.float32)),
    mesh=_mesh,
    scratch_types=[
        pltpu.VMEM((NCHUNK, CH), jnp.int32),
        pltpu.VMEM((NCHUNK, CH), jnp.int32),
        pltpu.VMEM((NB, CH, H), jnp.float32),
        pltpu.VMEM_SHARED((VP, H), jnp.float32),
        pltpu.SemaphoreType.DMA((NB,)),
    ],
    compiler_params=pltpu.CompilerParams(use_tc_tiling_on_sc=False),
)
def _sc_edge_pass(a_hbm, src_hbm, dst_hbm, z_hbm, out0_hbm, out1_hbm,
                  srcv, dstv, rows, acc, gsem):
    c = lax.axis_index("c")
    s = lax.axis_index("s")
    wid = c * NS + s

    pltpu.sync_copy(z_hbm.at[pl.ds(s * RPS, RPS)],
                    acc.at[pl.ds(s * RPS, RPS)])
    pltpu.sync_copy(src_hbm.at[wid], srcv)
    pltpu.sync_copy(dst_hbm.at[wid], dstv)
    plsc.subcore_barrier()

    def _start_gather(j, b):
        pltpu.async_copy(a_hbm.at[srcv.at[j]], rows.at[b], gsem.at[b])

    def _wait_gather(b):
        pltpu.make_async_copy(
            a_hbm.at[srcv.at[0]], rows.at[b], gsem.at[b]).wait()

    def _scatter(j, b):
        pltpu.sync_copy(rows.at[b], acc.at[dstv.at[j]], add=True)

    for b in range(NB):
        _start_gather(b, b)

    _MAIN = (NCHUNK // NB) * NB
    _TAIL = NCHUNK - _MAIN

    @pl.loop(0, _MAIN, step=NB)
    def _(j):
        for b in range(NB):
            _wait_gather(b)
            _scatter(j + b, b)

            @pl.when(j + b + NB < NCHUNK)
            def _():
                _start_gather(j + b + NB, b)

    for b in range(_TAIL):
        _wait_gather(b)
        _scatter(_MAIN + b, b)

    plsc.subcore_barrier()

    @pl.when(c == 0)
    def _():
        pltpu.sync_copy(acc.at[pl.ds(s * RPS, RPS)],
                        out0_hbm.at[pl.ds(s * RPS, RPS)])

    @pl.when(c == 1)
    def _():
        pltpu.sync_copy(acc.at[pl.ds(s * RPS, RPS)],
                        out1_hbm.at[pl.ds(s * RPS, RPS)])


def _blockdiag(w):
    n, m = w.shape
    z = jnp.zeros((n, m), w.dtype)
    return jnp.concatenate(
        [jnp.concatenate([w, z], axis=1), jnp.concatenate([z, w], axis=1)],
        axis=0)


def kernel(node_features, adjacency_list_0, node_to_graph_map, num_graphs,
           W_init, W_mp0, W_mp1, W_mp2, W_mp3, W_dense0, W_dense2):
    src3 = adjacency_list_0[:, 0].reshape(NW, NCHUNK, CH)
    dst3 = adjacency_list_0[:, 1].reshape(NW, NCHUNK, CH)
    zeros = jnp.zeros((VP, H), jnp.float32)
    x2 = jnp.pad(node_features, ((0, VP - V), (0, 0))).reshape(PR, 2 * D)

    wi2 = _blockdiag(W_init)
    wmp0_2 = _blockdiag(W_mp0)
    wmp1_2 = _blockdiag(W_mp1)
    wmp2_2 = _blockdiag(W_mp2)
    wmp3_2 = _blockdiag(W_mp3)
    wd0_2 = _blockdiag(W_dense0)
    wd2_2 = _blockdiag(W_dense2)

    def edge_pass(a_flat):
        pa, pb = _sc_edge_pass(a_flat.reshape(VP, H), src3, dst3, zeros)
        return pa.reshape(VPH), pb.reshape(VPH)

    h0, a0 = _tc0(x2, wi2, wmp0_2)
    p0a, p0b = edge_pass(a0)
    a1 = _tc_dense(p0a, p0b, wd0_2, wmp1_2)
    p1a, p1b = edge_pass(a1)
    a2 = _tc_res(p1a, p1b, h0, wmp2_2)
    p2a, p2b = edge_pass(a2)
    a3 = _tc_dense(p2a, p2b, wd2_2, wmp3_2)
    p3a, p3b = edge_pass(a3)
    out_flat = _tc_sum(p3a, p3b)
    return out_flat.reshape(VP, H)[:V]


# speedup vs baseline: 2.5956x; 1.0104x over previous
import functools

import jax
import jax.numpy as jnp
from jax import lax
from jax.experimental import pallas as pl
from jax.experimental.pallas import tpu as pltpu
from jax.experimental.pallas import tpu_sc as plsc

V = 10000
D = 128
H = 64
E = 320000
NC = 2
NS = 16
NW = NC * NS
EPW = E // NW
CH = 80
NCHUNK = EPW // CH
NB = 10
RPS = V // NS

VH = V * H
PR = V // 2


def _mm(x, w):
    return jnp.dot(x, w, preferred_element_type=jnp.float32)


def _pairs(ref):
    return ref[...].reshape(PR, 2 * H)


def _t0_body(x2_ref, wi_ref, wm_ref, h_ref, a_ref):
    h = jnp.tanh(_mm(x2_ref[...], wi_ref[...]))
    h_ref[...] = h.reshape(VH)
    a_ref[...] = jnp.maximum(_mm(h, wm_ref[...]), 0.0).reshape(VH)


def _t_dense_body(p0_ref, p1_ref, wd_ref, wm_ref, a_ref):
    s = _pairs(p0_ref) + _pairs(p1_ref)
    c = jnp.tanh(_mm(s, wd_ref[...]))
    a_ref[...] = jnp.maximum(_mm(c, wm_ref[...]), 0.0).reshape(VH)


def _t_res_body(p0_ref, p1_ref, h_ref, wm_ref, a_ref):
    m = (_pairs(p0_ref) + _pairs(p1_ref) + _pairs(h_ref)) * 0.5
    a_ref[...] = jnp.maximum(_mm(m, wm_ref[...]), 0.0).reshape(VH)


def _t_sum_body(p0_ref, p1_ref, o_ref):
    o_ref[...] = p0_ref[...] + p1_ref[...]


_F = jax.ShapeDtypeStruct


def _tc0(x2, wi2, wm2):
    return pl.pallas_call(
        _t0_body,
        out_shape=(_F((VH,), jnp.float32), _F((VH,), jnp.float32)),
    )(x2, wi2, wm2)


def _tc_dense(p0, p1, wd2, wm2):
    return pl.pallas_call(
        _t_dense_body, out_shape=_F((VH,), jnp.float32))(p0, p1, wd2, wm2)


def _tc_res(p0, p1, h, wm2):
    return pl.pallas_call(
        _t_res_body, out_shape=_F((VH,), jnp.float32))(p0, p1, h, wm2)


def _tc_sum(p0, p1):
    return pl.pallas_call(
        _t_sum_body, out_shape=_F((VH,), jnp.float32))(p0, p1)


_mesh = plsc.VectorSubcoreMesh(core_axis_name="c", subcore_axis_name="s")


@functools.partial(
    pl.kernel,
    out_type=(_F((V, H), jnp.float32), _F((V, H), jnp.float32)),
    mesh=_mesh,
    scratch_types=[
        pltpu.VMEM((NCHUNK, CH), jnp.int32),
        pltpu.VMEM((NCHUNK, CH), jnp.int32),
        pltpu.VMEM((NB, CH, H), jnp.float32),
        pltpu.VMEM_SHARED((V, H), jnp.float32),
        pltpu.SemaphoreType.DMA((NB,)),
    ],
    compiler_params=pltpu.CompilerParams(use_tc_tiling_on_sc=False),
)
def _sc_edge_pass(a_hbm, src_hbm, dst_hbm, z_hbm, out0_hbm, out1_hbm,
                  srcv, dstv, rows, acc, gsem):
    c = lax.axis_index("c")
    s = lax.axis_index("s")
    wid = c * NS + s

    pltpu.sync_copy(z_hbm.at[pl.ds(s * RPS, RPS)],
                    acc.at[pl.ds(s * RPS, RPS)])
    pltpu.sync_copy(src_hbm.at[wid], srcv)
    pltpu.sync_copy(dst_hbm.at[wid], dstv)
    plsc.subcore_barrier()

    def _start_gather(j, b):
        pltpu.async_copy(a_hbm.at[srcv.at[j]], rows.at[b], gsem.at[b])

    def _wait_gather(b):
        pltpu.make_async_copy(
            a_hbm.at[srcv.at[0]], rows.at[b], gsem.at[b]).wait()

    def _scatter(j, b):
        pltpu.sync_copy(rows.at[b], acc.at[dstv.at[j]], add=True)

    for b in range(NB):
        _start_gather(b, b)

    _MAIN = (NCHUNK // NB) * NB
    _TAIL = NCHUNK - _MAIN

    @pl.loop(0, _MAIN, step=NB)
    def _(j):
        for b in range(NB):
            _wait_gather(b)
            _scatter(j + b, b)

            @pl.when(j + b + NB < NCHUNK)
            def _():
                _start_gather(j + b + NB, b)

    for b in range(_TAIL):
        _wait_gather(b)
        _scatter(_MAIN + b, b)

    plsc.subcore_barrier()

    @pl.when(c == 0)
    def _():
        pltpu.sync_copy(acc.at[pl.ds(s * RPS, RPS)],
                        out0_hbm.at[pl.ds(s * RPS, RPS)])

    @pl.when(c == 1)
    def _():
        pltpu.sync_copy(acc.at[pl.ds(s * RPS, RPS)],
                        out1_hbm.at[pl.ds(s * RPS, RPS)])


def _blockdiag(w):
    n, m = w.shape
    z = jnp.zeros((n, m), w.dtype)
    return jnp.concatenate(
        [jnp.concatenate([w, z], axis=1), jnp.concatenate([z, w], axis=1)],
        axis=0)


def kernel(node_features, adjacency_list_0, node_to_graph_map, num_graphs,
           W_init, W_mp0, W_mp1, W_mp2, W_mp3, W_dense0, W_dense2):
    src3 = adjacency_list_0[:, 0].reshape(NW, NCHUNK, CH)
    dst3 = adjacency_list_0[:, 1].reshape(NW, NCHUNK, CH)
    zeros = jnp.zeros((V, H), jnp.float32)
    x2 = node_features.reshape(PR, 2 * D)

    wi2 = _blockdiag(W_init)
    wmp0_2 = _blockdiag(W_mp0)
    wmp1_2 = _blockdiag(W_mp1)
    wmp2_2 = _blockdiag(W_mp2)
    wmp3_2 = _blockdiag(W_mp3)
    wd0_2 = _blockdiag(W_dense0)
    wd2_2 = _blockdiag(W_dense2)

    def edge_pass(a_flat):
        pa, pb = _sc_edge_pass(a_flat.reshape(V, H), src3, dst3, zeros)
        return pa.reshape(VH), pb.reshape(VH)

    h0, a0 = _tc0(x2, wi2, wmp0_2)
    p0a, p0b = edge_pass(a0)
    a1 = _tc_dense(p0a, p0b, wd0_2, wmp1_2)
    p1a, p1b = edge_pass(a1)
    a2 = _tc_res(p1a, p1b, h0, wmp2_2)
    p2a, p2b = edge_pass(a2)
    a3 = _tc_dense(p2a, p2b, wd2_2, wmp3_2)
    p3a, p3b = edge_pass(a3)
    out_flat = _tc_sum(p3a, p3b)
    return out_flat.reshape(V, H)[:V]
